# baseline (speedup 1.0000x reference)
import functools

import jax
import jax.numpy as jnp
from jax import lax
from jax.experimental import pallas as pl
from jax.experimental.pallas import tpu as pltpu
from jax.experimental.pallas import tpu_sc as plsc

B = 4
N = 8192
NPOINT = 1024
NSAMPLE = 32
C = 128
R2 = 0.2 * 0.2

NC = 2
NS = 16
L = 16
NW = NC * NS
CPT = (B * NPOINT) // NW
TPB = NW // B
CHT = C // TPB
NCHUNK = N // L
CB = 4
RB = 256
NFLAT = NPOINT * NSAMPLE

_mesh = plsc.VectorSubcoreMesh(
    core_axis_name="c", subcore_axis_name="s", num_cores=NC, num_subcores=NS)
_params = pltpu.CompilerParams(use_tc_tiling_on_sc=False,
                               needs_layout_passes=False)


def _bf16_round(x):
    bits = plsc.bitcast(x, jnp.int32)
    lsb = (bits >> 16) & 1
    rb = (bits + 0x7FFF + lsb) & jnp.int32(-65536)
    return plsc.bitcast(rb, jnp.float32)


@functools.partial(
    pl.kernel,
    out_type=(
        jax.ShapeDtypeStruct((B, NFLAT), jnp.int32),
        jax.ShapeDtypeStruct((B, 3, NFLAT), jnp.float32),
    ),
    mesh=_mesh,
    compiler_params=_params,
    scratch_types=[
        pltpu.VMEM((N, 3), jnp.float32),
        pltpu.VMEM((N,), jnp.float32),
        pltpu.VMEM((N,), jnp.float32),
        pltpu.VMEM((N,), jnp.float32),
        pltpu.VMEM((N,), jnp.float32),
        pltpu.VMEM((CPT, 3), jnp.float32),
        pltpu.VMEM((CPT * NSAMPLE,), jnp.int32),
        pltpu.VMEM((3, CPT * NSAMPLE), jnp.float32),
    ],
)
def _ball_query_kernel(xyz, new_xyz, idx_out, gxyz_out,
                       pts, spv, x2b, y2b, z2b, ctrs, selbuf, xyzbuf):
    ci = lax.axis_index("c")
    si = lax.axis_index("s")
    wid = ci * NS + si
    b = wid // TPB
    m0 = (wid % TPB) * CPT

    pltpu.sync_copy(xyz.at[b, pl.ds(0, N), pl.ds(0, 3)], pts)
    pltpu.sync_copy(new_xyz.at[b, pl.ds(m0, CPT), pl.ds(0, 3)], ctrs)

    iot = lax.iota(jnp.int32, L)
    zz = jnp.zeros((L,), jnp.int32)
    one = zz + 1
    two = zz + 2
    for w in range(CPT // L):
        plsc.store_scatter(selbuf, [(w * L + iot) * NSAMPLE], zz)

    def pre_round(i, carry):
        base = i * L
        bi = base + iot
        xs = plsc.load_gather(pts, [bi, zz])
        ys = plsc.load_gather(pts, [bi, one])
        zs = plsc.load_gather(pts, [bi, two])
        spv[pl.ds(base, L)] = (xs * xs + ys * ys) + zs * zs
        x2b[pl.ds(base, L)] = 2.0 * _bf16_round(xs)
        y2b[pl.ds(base, L)] = 2.0 * _bf16_round(ys)
        z2b[pl.ds(base, L)] = 2.0 * _bf16_round(zs)
        return carry

    lax.fori_loop(0, NCHUNK, pre_round, 0)

    def per_center(mi, carry):
        msp = jnp.full((L,), mi, jnp.int32)
        s0 = mi * NSAMPLE
        s0v = jnp.full((L,), s0, jnp.int32)
        cx = plsc.load_gather(ctrs, [msp, zz])
        cy = plsc.load_gather(ctrs, [msp, one])
        cz = plsc.load_gather(ctrs, [msp, two])
        sc = (cx * cx + cy * cy) + cz * cz
        cxb = _bf16_round(cx)
        cyb = _bf16_round(cy)
        czb = _bf16_round(cz)

        def cond(st):
            i, cnt = st
            return jnp.logical_and(i < NCHUNK // 2, cnt < NSAMPLE)

        def body(st):
            i, cnt = st
            base = i * (2 * L)
            b2 = base + L
            d21 = (sc + spv[pl.ds(base, L)]) - (
                (cxb * x2b[pl.ds(base, L)] + cyb * y2b[pl.ds(base, L)])
                + czb * z2b[pl.ds(base, L)])
            d22 = (sc + spv[pl.ds(b2, L)]) - (
                (cxb * x2b[pl.ds(b2, L)] + cyb * y2b[pl.ds(b2, L)])
                + czb * z2b[pl.ds(b2, L)])
            m1 = d21 < R2
            m2 = d22 < R2
            p1 = plsc.all_reduce_population_count(m1)
            p2 = plsc.all_reduce_population_count(m2)
            cntv = jnp.full((L,), cnt, jnp.int32)
            inc1 = plsc.cumsum(m1.astype(jnp.int32))
            slot1 = (cntv + inc1) - 1
            wm1 = jnp.logical_and(m1, slot1 < NSAMPLE)
            plsc.store_scatter(selbuf, [s0v + slot1], base + iot, mask=wm1)
            inc2 = plsc.cumsum(m2.astype(jnp.int32))
            slot2 = ((cntv + p1) + inc2) - 1
            wm2 = jnp.logical_and(m2, slot2 < NSAMPLE)
            plsc.store_scatter(selbuf, [s0v + slot2], b2 + iot, mask=wm2)
            tot = p1 + p2
            return (i + 1, cnt + tot[0])

        _, cnt = lax.while_loop(cond, body, (jnp.int32(0), jnp.int32(0)))

        firstv = plsc.load_gather(selbuf, [s0v])
        cntv = jnp.full((L,), cnt, jnp.int32)
        for h in range(NSAMPLE // L):
            jv = h * L + iot
            cur = selbuf[pl.ds(s0 + h * L, L)]
            selv = jnp.where(jv >= cntv, firstv, cur)
            selbuf[pl.ds(s0 + h * L, L)] = selv
            gx = plsc.load_gather(pts, [selv, zz]) - cx
            gy = plsc.load_gather(pts, [selv, one]) - cy
            gz = plsc.load_gather(pts, [selv, two]) - cz
            xyzbuf[0, pl.ds(s0 + h * L, L)] = gx
            xyzbuf[1, pl.ds(s0 + h * L, L)] = gy
            xyzbuf[2, pl.ds(s0 + h * L, L)] = gz
        return carry

    lax.fori_loop(0, CPT, per_center, 0)

    pltpu.sync_copy(selbuf, idx_out.at[b, pl.ds(m0 * NSAMPLE, CPT * NSAMPLE)])
    pltpu.sync_copy(xyzbuf,
                    gxyz_out.at[b, pl.ds(0, 3), pl.ds(m0 * NSAMPLE, CPT * NSAMPLE)])


@functools.partial(
    pl.kernel,
    out_type=jax.ShapeDtypeStruct((B, 3 + C, NFLAT), jnp.float32),
    mesh=_mesh,
    compiler_params=_params,
    scratch_types=[
        pltpu.VMEM((NFLAT,), jnp.int32),
        pltpu.VMEM((N,), jnp.float32),
        pltpu.VMEM((N,), jnp.float32),
        pltpu.VMEM((N,), jnp.float32),
        pltpu.VMEM((N,), jnp.float32),
        pltpu.VMEM((RB * NSAMPLE,), jnp.float32),
        pltpu.VMEM((RB * NSAMPLE,), jnp.float32),
        pltpu.VMEM((RB * NSAMPLE,), jnp.float32),
        pltpu.VMEM((RB * NSAMPLE,), jnp.float32),
    ],
)
def _group_kernel(features, idxq, gxyz, out, idx_f,
                  frow0, frow1, frow2, frow3, ob0, ob1, ob2, ob3):
    ci = lax.axis_index("c")
    si = lax.axis_index("s")
    wid = ci * NS + si
    b = wid // TPB
    c0 = (wid % TPB) * CHT
    frows = (frow0, frow1, frow2, frow3)
    obufs = (ob0, ob1, ob2, ob3)

    pltpu.sync_copy(idxq.at[b, pl.ds(0, NFLAT)], idx_f)

    @pl.when(wid % TPB == 0)
    def _copy_xyz():
        pltpu.sync_copy(gxyz.at[b, pl.ds(0, 3), pl.ds(0, NFLAT)],
                        out.at[b, pl.ds(0, 3), pl.ds(0, NFLAT)])

    for cb in range(CHT // CB):
        c = c0 + cb * CB
        for j in range(CB):
            pltpu.sync_copy(features.at[b, c + j, pl.ds(0, N)], frows[j])

        for rb in range(NPOINT // RB):
            def per_row(ri, carry, rb=rb):
                g = rb * RB * NSAMPLE + ri * NSAMPLE
                o = ri * NSAMPLE
                for h in range(NSAMPLE // L):
                    iv = idx_f[pl.ds(g + h * L, L)]
                    for j in range(CB):
                        obufs[j][pl.ds(o + h * L, L)] = \
                            plsc.load_gather(frows[j], [iv])
                return carry

            lax.fori_loop(0, RB, per_row, 0)
            for j in range(CB):
                pltpu.sync_copy(
                    obufs[j],
                    out.at[b, 3 + c + j,
                           pl.ds(rb * RB * NSAMPLE, RB * NSAMPLE)])


def kernel(xyz, new_xyz, features):
    idxq, gxyz = _ball_query_kernel(xyz, new_xyz)
    out = _group_kernel(features, idxq, gxyz)
    return out.reshape(B, 3 + C, NPOINT, NSAMPLE)


# speedup vs baseline: 1.7477x; 1.7477x over previous
import functools

import jax
import jax.numpy as jnp
from jax import lax
from jax.experimental import pallas as pl
from jax.experimental.pallas import tpu as pltpu
from jax.experimental.pallas import tpu_sc as plsc

B = 4
N = 8192
NPOINT = 1024
NSAMPLE = 32
C = 128
R2 = 0.2 * 0.2

NC = 2
NS = 16
L = 16
NW = NC * NS
CPT = (B * NPOINT) // NW
TPB = NW // B
CHT = C // TPB
NCHUNK = N // L
CB = 4
RB = 256

_mesh = plsc.VectorSubcoreMesh(
    core_axis_name="c", subcore_axis_name="s", num_cores=NC, num_subcores=NS)
_params = pltpu.CompilerParams(use_tc_tiling_on_sc=False,
                               needs_layout_passes=False)


def _bf16_round(x):
    bits = plsc.bitcast(x, jnp.int32)
    lsb = (bits >> 16) & 1
    rb = (bits + 0x7FFF + lsb) & jnp.int32(-65536)
    return plsc.bitcast(rb, jnp.float32)


@functools.partial(
    pl.kernel,
    out_type=jax.ShapeDtypeStruct((B, 3 + C, NPOINT, NSAMPLE), jnp.float32),
    mesh=_mesh,
    compiler_params=_params,
    scratch_types=[
        pltpu.VMEM_SHARED((2, NPOINT, NSAMPLE), jnp.int32),
    ],
)
def _qag_kernel(xyz, new_xyz, features, out, shared_idx):
    ci = lax.axis_index("c")
    si = lax.axis_index("s")
    wid = ci * NS + si
    b = wid // TPB
    lb = b % 2
    m0 = (wid % TPB) * CPT

    iot = lax.iota(jnp.int32, L)
    zz = jnp.zeros((L,), jnp.int32)
    one = zz + 1
    two = zz + 2

    def phase_a(pts, spv, x2b, y2b, z2b, ctrs, selbuf, xyzbuf):
        pltpu.sync_copy(xyz.at[b, pl.ds(0, N), pl.ds(0, 3)], pts)
        pltpu.sync_copy(new_xyz.at[b, pl.ds(m0, CPT), pl.ds(0, 3)], ctrs)

        for w in range(CPT // L):
            plsc.store_scatter(selbuf, [w * L + iot, zz], zz)

        def pre_round(i, carry):
            base = i * L
            bi = base + iot
            xs = plsc.load_gather(pts, [bi, zz])
            ys = plsc.load_gather(pts, [bi, one])
            zs = plsc.load_gather(pts, [bi, two])
            spv[pl.ds(base, L)] = (xs * xs + ys * ys) + zs * zs
            x2b[pl.ds(base, L)] = 2.0 * _bf16_round(xs)
            y2b[pl.ds(base, L)] = 2.0 * _bf16_round(ys)
            z2b[pl.ds(base, L)] = 2.0 * _bf16_round(zs)
            return carry

        lax.fori_loop(0, NCHUNK, pre_round, 0)

        def per_center(mi, carry):
            msp = jnp.full((L,), mi, jnp.int32)
            cx = plsc.load_gather(ctrs, [msp, zz])
            cy = plsc.load_gather(ctrs, [msp, one])
            cz = plsc.load_gather(ctrs, [msp, two])
            sc = (cx * cx + cy * cy) + cz * cz
            cxb = _bf16_round(cx)
            cyb = _bf16_round(cy)
            czb = _bf16_round(cz)

            def cond(st):
                i, cnt = st
                return jnp.logical_and(i < NCHUNK // 2, cnt < NSAMPLE)

            def body(st):
                i, cnt = st
                base = i * (2 * L)
                b2 = base + L
                d21 = (sc + spv[pl.ds(base, L)]) - (
                    (cxb * x2b[pl.ds(base, L)] + cyb * y2b[pl.ds(base, L)])
                    + czb * z2b[pl.ds(base, L)])
                d22 = (sc + spv[pl.ds(b2, L)]) - (
                    (cxb * x2b[pl.ds(b2, L)] + cyb * y2b[pl.ds(b2, L)])
                    + czb * z2b[pl.ds(b2, L)])
                m1 = d21 < R2
                m2 = d22 < R2
                p1 = plsc.all_reduce_population_count(m1)
                p2 = plsc.all_reduce_population_count(m2)
                cntv = jnp.full((L,), cnt, jnp.int32)
                inc1 = plsc.cumsum(m1.astype(jnp.int32))
                slot1 = (cntv + inc1) - 1
                wm1 = jnp.logical_and(m1, slot1 < NSAMPLE)
                plsc.store_scatter(selbuf, [msp, slot1], base + iot, mask=wm1)
                inc2 = plsc.cumsum(m2.astype(jnp.int32))
                slot2 = ((cntv + p1) + inc2) - 1
                wm2 = jnp.logical_and(m2, slot2 < NSAMPLE)
                plsc.store_scatter(selbuf, [msp, slot2], b2 + iot, mask=wm2)
                tot = p1 + p2
                return (i + 1, cnt + tot[0])

            _, cnt = lax.while_loop(cond, body, (jnp.int32(0), jnp.int32(0)))

            firstv = plsc.load_gather(selbuf, [msp, zz])
            cntv = jnp.full((L,), cnt, jnp.int32)
            for h in range(NSAMPLE // L):
                jv = h * L + iot
                cur = plsc.load_gather(selbuf, [msp, jv])
                selv = jnp.where(jv >= cntv, firstv, cur)
                plsc.store_scatter(selbuf, [msp, jv], selv)
                gx = plsc.load_gather(pts, [selv, zz]) - cx
                gy = plsc.load_gather(pts, [selv, one]) - cy
                gz = plsc.load_gather(pts, [selv, two]) - cz
                plsc.store_scatter(xyzbuf, [zz, msp, jv], gx)
                plsc.store_scatter(xyzbuf, [one, msp, jv], gy)
                plsc.store_scatter(xyzbuf, [two, msp, jv], gz)
            return carry

        lax.fori_loop(0, CPT, per_center, 0)

        pltpu.sync_copy(selbuf,
                        shared_idx.at[lb, pl.ds(m0, CPT), pl.ds(0, NSAMPLE)])
        pltpu.sync_copy(
            xyzbuf,
            out.at[b, pl.ds(0, 3), pl.ds(m0, CPT), pl.ds(0, NSAMPLE)])

    pl.run_scoped(
        phase_a,
        pltpu.VMEM((N, 3), jnp.float32),
        pltpu.VMEM((N,), jnp.float32),
        pltpu.VMEM((N,), jnp.float32),
        pltpu.VMEM((N,), jnp.float32),
        pltpu.VMEM((N,), jnp.float32),
        pltpu.VMEM((CPT, 3), jnp.float32),
        pltpu.VMEM((CPT, NSAMPLE), jnp.int32),
        pltpu.VMEM((3, CPT, NSAMPLE), jnp.float32),
    )

    plsc.subcore_barrier()

    c0 = (wid % TPB) * CHT
    jsp = [jnp.full((L,), j, jnp.int32) for j in range(CB)]

    def phase_b(idx_s, frow0, frow1, frow2, frow3, ob0, ob1, ob2, ob3):
        frows = (frow0, frow1, frow2, frow3)
        obufs = (ob0, ob1, ob2, ob3)
        pltpu.sync_copy(
            shared_idx.at[lb, pl.ds(0, NPOINT), pl.ds(0, NSAMPLE)], idx_s)

        for cb in range(CHT // CB):
            c = c0 + cb * CB
            for j in range(CB):
                pltpu.sync_copy(features.at[b, c + j, pl.ds(0, N)], frows[j])

            for rb in range(NPOINT // RB):
                def per_row(ri, carry, rb=rb):
                    rv = jnp.full((L,), rb * RB + ri, jnp.int32)
                    riv = jnp.full((L,), ri, jnp.int32)
                    for h in range(NSAMPLE // L):
                        cv = h * L + iot
                        iv = plsc.load_gather(idx_s, [rv, cv])
                        for j in range(CB):
                            vals = plsc.load_gather(frows[j], [iv])
                            plsc.store_scatter(obufs[j], [riv, cv], vals)
                    return carry

                lax.fori_loop(0, RB, per_row, 0)
                for j in range(CB):
                    pltpu.sync_copy(
                        obufs[j],
                        out.at[b, 3 + c + j, pl.ds(rb * RB, RB),
                               pl.ds(0, NSAMPLE)])

    pl.run_scoped(
        phase_b,
        pltpu.VMEM((NPOINT, NSAMPLE), jnp.int32),
        pltpu.VMEM((N,), jnp.float32),
        pltpu.VMEM((N,), jnp.float32),
        pltpu.VMEM((N,), jnp.float32),
        pltpu.VMEM((N,), jnp.float32),
        pltpu.VMEM((RB, NSAMPLE), jnp.float32),
        pltpu.VMEM((RB, NSAMPLE), jnp.float32),
        pltpu.VMEM((RB, NSAMPLE), jnp.float32),
        pltpu.VMEM((RB, NSAMPLE), jnp.float32),
    )


def kernel(xyz, new_xyz, features):
    return _qag_kernel(xyz, new_xyz, features)


# speedup vs baseline: 1.7515x; 1.0021x over previous
import functools

import jax
import jax.numpy as jnp
from jax import lax
from jax.experimental import pallas as pl
from jax.experimental.pallas import tpu as pltpu
from jax.experimental.pallas import tpu_sc as plsc

B = 4
N = 8192
NPOINT = 1024
NSAMPLE = 32
C = 128
R2 = 0.2 * 0.2

NC = 2
NS = 16
L = 16
NW = NC * NS
CPT = (B * NPOINT) // NW
TPB = NW // B
CHT = C // TPB
NCHUNK = N // L
CB = 4
RB = 256

_mesh = plsc.VectorSubcoreMesh(
    core_axis_name="c", subcore_axis_name="s", num_cores=NC, num_subcores=NS)
_params = pltpu.CompilerParams(use_tc_tiling_on_sc=False,
                               needs_layout_passes=False)


def _bf16_round(x):
    bits = plsc.bitcast(x, jnp.int32)
    lsb = (bits >> 16) & 1
    rb = (bits + 0x7FFF + lsb) & jnp.int32(-65536)
    return plsc.bitcast(rb, jnp.float32)


@functools.partial(
    pl.kernel,
    out_type=jax.ShapeDtypeStruct((B, 3 + C, NPOINT, NSAMPLE), jnp.float32),
    mesh=_mesh,
    compiler_params=_params,
    scratch_types=[
        pltpu.VMEM_SHARED((2, NPOINT, NSAMPLE), jnp.int32),
    ],
)
def _qag_kernel(xyz, new_xyz, features, out, shared_idx):
    ci = lax.axis_index("c")
    si = lax.axis_index("s")
    wid = ci * NS + si
    b = wid // TPB
    lb = b % 2
    m0 = (wid % TPB) * CPT

    iot = lax.iota(jnp.int32, L)
    zz = jnp.zeros((L,), jnp.int32)
    one = zz + 1
    two = zz + 2

    def phase_a(pts, spv, x2b, y2b, z2b, ctrs, selbuf, xyzbuf):
        pltpu.sync_copy(xyz.at[b, pl.ds(0, N), pl.ds(0, 3)], pts)
        pltpu.sync_copy(new_xyz.at[b, pl.ds(m0, CPT), pl.ds(0, 3)], ctrs)

        for w in range(CPT // L):
            plsc.store_scatter(selbuf, [w * L + iot, zz], zz)

        def pre_round(i, carry):
            base = i * L
            bi = base + iot
            xs = plsc.load_gather(pts, [bi, zz])
            ys = plsc.load_gather(pts, [bi, one])
            zs = plsc.load_gather(pts, [bi, two])
            spv[pl.ds(base, L)] = (xs * xs + ys * ys) + zs * zs
            x2b[pl.ds(base, L)] = 2.0 * _bf16_round(xs)
            y2b[pl.ds(base, L)] = 2.0 * _bf16_round(ys)
            z2b[pl.ds(base, L)] = 2.0 * _bf16_round(zs)
            return carry

        lax.fori_loop(0, NCHUNK, pre_round, 0)

        def per_center(mi, carry):
            msp = jnp.full((L,), mi, jnp.int32)
            cx = plsc.load_gather(ctrs, [msp, zz])
            cy = plsc.load_gather(ctrs, [msp, one])
            cz = plsc.load_gather(ctrs, [msp, two])
            sc = (cx * cx + cy * cy) + cz * cz
            cxb = _bf16_round(cx)
            cyb = _bf16_round(cy)
            czb = _bf16_round(cz)

            def cond(st):
                i, cnt = st
                return jnp.logical_and(i < NCHUNK // 2, cnt < NSAMPLE)

            def body(st):
                i, cnt = st
                base = i * (2 * L)
                b2 = base + L
                d21 = (sc + spv[pl.ds(base, L)]) - (
                    (cxb * x2b[pl.ds(base, L)] + cyb * y2b[pl.ds(base, L)])
                    + czb * z2b[pl.ds(base, L)])
                d22 = (sc + spv[pl.ds(b2, L)]) - (
                    (cxb * x2b[pl.ds(b2, L)] + cyb * y2b[pl.ds(b2, L)])
                    + czb * z2b[pl.ds(b2, L)])
                m1 = d21 < R2
                m2 = d22 < R2
                p1 = plsc.all_reduce_population_count(m1)
                p2 = plsc.all_reduce_population_count(m2)
                cntv = jnp.full((L,), cnt, jnp.int32)
                inc1 = plsc.cumsum(m1.astype(jnp.int32))
                slot1 = (cntv + inc1) - 1
                wm1 = jnp.logical_and(m1, slot1 < NSAMPLE)
                plsc.store_scatter(selbuf, [msp, slot1], base + iot, mask=wm1)
                inc2 = plsc.cumsum(m2.astype(jnp.int32))
                slot2 = ((cntv + p1) + inc2) - 1
                wm2 = jnp.logical_and(m2, slot2 < NSAMPLE)
                plsc.store_scatter(selbuf, [msp, slot2], b2 + iot, mask=wm2)
                tot = p1 + p2
                return (i + 1, cnt + tot[0])

            _, cnt = lax.while_loop(cond, body, (jnp.int32(0), jnp.int32(0)))

            firstv = plsc.load_gather(selbuf, [msp, zz])
            cntv = jnp.full((L,), cnt, jnp.int32)
            for h in range(NSAMPLE // L):
                jv = h * L + iot
                cur = plsc.load_gather(selbuf, [msp, jv])
                selv = jnp.where(jv >= cntv, firstv, cur)
                plsc.store_scatter(selbuf, [msp, jv], selv)
                gx = plsc.load_gather(pts, [selv, zz]) - cx
                gy = plsc.load_gather(pts, [selv, one]) - cy
                gz = plsc.load_gather(pts, [selv, two]) - cz
                plsc.store_scatter(xyzbuf, [zz, msp, jv], gx)
                plsc.store_scatter(xyzbuf, [one, msp, jv], gy)
                plsc.store_scatter(xyzbuf, [two, msp, jv], gz)
            return carry

        lax.fori_loop(0, CPT, per_center, 0)

        pltpu.sync_copy(selbuf,
                        shared_idx.at[lb, pl.ds(m0, CPT), pl.ds(0, NSAMPLE)])
        pltpu.sync_copy(
            xyzbuf,
            out.at[b, pl.ds(0, 3), pl.ds(m0, CPT), pl.ds(0, NSAMPLE)])

    pl.run_scoped(
        phase_a,
        pltpu.VMEM((N, 3), jnp.float32),
        pltpu.VMEM((N,), jnp.float32),
        pltpu.VMEM((N,), jnp.float32),
        pltpu.VMEM((N,), jnp.float32),
        pltpu.VMEM((N,), jnp.float32),
        pltpu.VMEM((CPT, 3), jnp.float32),
        pltpu.VMEM((CPT, NSAMPLE), jnp.int32),
        pltpu.VMEM((3, CPT, NSAMPLE), jnp.float32),
    )

    plsc.subcore_barrier()

    c0 = (wid % TPB) * CHT
    jsp = [jnp.full((L,), j, jnp.int32) for j in range(CB)]

    def phase_b(idx_s, frow0, frow1, frow2, frow3, ob0, ob1, ob2, ob3):
        frows = (frow0, frow1, frow2, frow3)
        obufs = (ob0, ob1, ob2, ob3)
        pltpu.sync_copy(
            shared_idx.at[lb, pl.ds(0, NPOINT), pl.ds(0, NSAMPLE)], idx_s)

        for cb in range(CHT // CB):
            c = c0 + cb * CB
            for j in range(CB):
                ch = c + j
                pltpu.sync_copy(
                    features.at[b, ch // 8, ch % 8, pl.ds(0, N // 128),
                                pl.ds(0, 128)],
                    frows[j])

            for rb in range(NPOINT // RB):
                def one_row(r, ri):
                    rv = jnp.full((L,), r, jnp.int32)
                    riv = jnp.full((L,), ri, jnp.int32)
                    for h in range(NSAMPLE // L):
                        cv = h * L + iot
                        iv = plsc.load_gather(idx_s, [rv, cv])
                        ihi = iv >> 7
                        ilo = iv & 127
                        for j in range(CB):
                            vals = plsc.load_gather(frows[j], [ihi, ilo])
                            plsc.store_scatter(obufs[j], [riv, cv], vals)

                def per_row(ri, carry, rb=rb):
                    r2i = 2 * ri
                    one_row(rb * RB + r2i, r2i)
                    one_row(rb * RB + r2i + 1, r2i + 1)
                    return carry

                lax.fori_loop(0, RB // 2, per_row, 0)
                for j in range(CB):
                    pltpu.sync_copy(
                        obufs[j],
                        out.at[b, 3 + c + j, pl.ds(rb * RB, RB),
                               pl.ds(0, NSAMPLE)])

    pl.run_scoped(
        phase_b,
        pltpu.VMEM((NPOINT, NSAMPLE), jnp.int32),
        pltpu.VMEM((N // 128, 128), jnp.float32),
        pltpu.VMEM((N // 128, 128), jnp.float32),
        pltpu.VMEM((N // 128, 128), jnp.float32),
        pltpu.VMEM((N // 128, 128), jnp.float32),
        pltpu.VMEM((RB, NSAMPLE), jnp.float32),
        pltpu.VMEM((RB, NSAMPLE), jnp.float32),
        pltpu.VMEM((RB, NSAMPLE), jnp.float32),
        pltpu.VMEM((RB, NSAMPLE), jnp.float32),
    )


def kernel(xyz, new_xyz, features):
    f5 = features.reshape(B, C // 8, 8, N // 128, 128)
    return _qag_kernel(xyz, new_xyz, f5)


# speedup vs baseline: 1.7965x; 1.0257x over previous
import functools

import jax
import jax.numpy as jnp
from jax import lax
from jax.experimental import pallas as pl
from jax.experimental.pallas import tpu as pltpu
from jax.experimental.pallas import tpu_sc as plsc

B = 4
N = 8192
NPOINT = 1024
NSAMPLE = 32
C = 128
R2 = 0.2 * 0.2

NC = 2
NS = 16
L = 16
NW = NC * NS
CPT = (B * NPOINT) // NW
TPB = NW // B
CHT = C // TPB
NCHUNK = N // L
CB = 4
RB = 128

_mesh = plsc.VectorSubcoreMesh(
    core_axis_name="c", subcore_axis_name="s", num_cores=NC, num_subcores=NS)
_params = pltpu.CompilerParams(use_tc_tiling_on_sc=False,
                               needs_layout_passes=False)


def _bf16_round(x):
    bits = plsc.bitcast(x, jnp.int32)
    lsb = (bits >> 16) & 1
    rb = (bits + 0x7FFF + lsb) & jnp.int32(-65536)
    return plsc.bitcast(rb, jnp.float32)


@functools.partial(
    pl.kernel,
    out_type=jax.ShapeDtypeStruct((B, 3 + C, NPOINT, NSAMPLE), jnp.float32),
    mesh=_mesh,
    compiler_params=_params,
    scratch_types=[
        pltpu.VMEM_SHARED((2, NPOINT, NSAMPLE), jnp.int32),
    ],
)
def _qag_kernel(xyz, new_xyz, features, out, shared_idx):
    ci = lax.axis_index("c")
    si = lax.axis_index("s")
    wid = ci * NS + si
    b = wid // TPB
    lb = b % 2
    m0 = (wid % TPB) * CPT

    iot = lax.iota(jnp.int32, L)
    zz = jnp.zeros((L,), jnp.int32)
    one = zz + 1
    two = zz + 2

    def phase_a(pts, spv, x2b, y2b, z2b, ctrs, selbuf, xyzbuf):
        pltpu.sync_copy(xyz.at[b, pl.ds(0, N), pl.ds(0, 3)], pts)
        pltpu.sync_copy(new_xyz.at[b, pl.ds(m0, CPT), pl.ds(0, 3)], ctrs)

        for w in range(CPT // L):
            plsc.store_scatter(selbuf, [w * L + iot, zz], zz)

        def pre_round(i, carry):
            base = i * L
            bi = base + iot
            xs = plsc.load_gather(pts, [bi, zz])
            ys = plsc.load_gather(pts, [bi, one])
            zs = plsc.load_gather(pts, [bi, two])
            spv[pl.ds(base, L)] = (xs * xs + ys * ys) + zs * zs
            x2b[pl.ds(base, L)] = 2.0 * _bf16_round(xs)
            y2b[pl.ds(base, L)] = 2.0 * _bf16_round(ys)
            z2b[pl.ds(base, L)] = 2.0 * _bf16_round(zs)
            return carry

        lax.fori_loop(0, NCHUNK, pre_round, 0)

        def per_center(mi, carry):
            msp = jnp.full((L,), mi, jnp.int32)
            cx = plsc.load_gather(ctrs, [msp, zz])
            cy = plsc.load_gather(ctrs, [msp, one])
            cz = plsc.load_gather(ctrs, [msp, two])
            sc = (cx * cx + cy * cy) + cz * cz
            cxb = _bf16_round(cx)
            cyb = _bf16_round(cy)
            czb = _bf16_round(cz)

            def cond(st):
                i, cnt = st
                return jnp.logical_and(i < NCHUNK // 2, cnt < NSAMPLE)

            def body(st):
                i, cnt = st
                base = i * (2 * L)
                b2 = base + L
                d21 = (sc + spv[pl.ds(base, L)]) - (
                    (cxb * x2b[pl.ds(base, L)] + cyb * y2b[pl.ds(base, L)])
                    + czb * z2b[pl.ds(base, L)])
                d22 = (sc + spv[pl.ds(b2, L)]) - (
                    (cxb * x2b[pl.ds(b2, L)] + cyb * y2b[pl.ds(b2, L)])
                    + czb * z2b[pl.ds(b2, L)])
                m1 = d21 < R2
                m2 = d22 < R2
                p1 = plsc.all_reduce_population_count(m1)
                p2 = plsc.all_reduce_population_count(m2)
                cntv = jnp.full((L,), cnt, jnp.int32)
                inc1 = plsc.cumsum(m1.astype(jnp.int32))
                slot1 = (cntv + inc1) - 1
                wm1 = jnp.logical_and(m1, slot1 < NSAMPLE)
                plsc.store_scatter(selbuf, [msp, slot1], base + iot, mask=wm1)
                inc2 = plsc.cumsum(m2.astype(jnp.int32))
                slot2 = ((cntv + p1) + inc2) - 1
                wm2 = jnp.logical_and(m2, slot2 < NSAMPLE)
                plsc.store_scatter(selbuf, [msp, slot2], b2 + iot, mask=wm2)
                tot = p1 + p2
                return (i + 1, cnt + tot[0])

            _, cnt = lax.while_loop(cond, body, (jnp.int32(0), jnp.int32(0)))

            firstv = plsc.load_gather(selbuf, [msp, zz])
            cntv = jnp.full((L,), cnt, jnp.int32)
            for h in range(NSAMPLE // L):
                jv = h * L + iot
                cur = plsc.load_gather(selbuf, [msp, jv])
                selv = jnp.where(jv >= cntv, firstv, cur)
                plsc.store_scatter(selbuf, [msp, jv], selv)
                gx = plsc.load_gather(pts, [selv, zz]) - cx
                gy = plsc.load_gather(pts, [selv, one]) - cy
                gz = plsc.load_gather(pts, [selv, two]) - cz
                plsc.store_scatter(xyzbuf, [zz, msp, jv], gx)
                plsc.store_scatter(xyzbuf, [one, msp, jv], gy)
                plsc.store_scatter(xyzbuf, [two, msp, jv], gz)
            return carry

        lax.fori_loop(0, CPT, per_center, 0)

        pltpu.sync_copy(selbuf,
                        shared_idx.at[lb, pl.ds(m0, CPT), pl.ds(0, NSAMPLE)])
        pltpu.sync_copy(
            xyzbuf,
            out.at[b, pl.ds(0, 3), pl.ds(m0, CPT), pl.ds(0, NSAMPLE)])

    pl.run_scoped(
        phase_a,
        pltpu.VMEM((N, 3), jnp.float32),
        pltpu.VMEM((N,), jnp.float32),
        pltpu.VMEM((N,), jnp.float32),
        pltpu.VMEM((N,), jnp.float32),
        pltpu.VMEM((N,), jnp.float32),
        pltpu.VMEM((CPT, 3), jnp.float32),
        pltpu.VMEM((CPT, NSAMPLE), jnp.int32),
        pltpu.VMEM((3, CPT, NSAMPLE), jnp.float32),
    )

    plsc.subcore_barrier()

    c0 = (wid % TPB) * CHT
    jsp = [jnp.full((L,), j, jnp.int32) for j in range(CB)]

    def phase_b(idx_s, frow0, frow1, frow2, frow3, obufs2, sems):
        frows = (frow0, frow1, frow2, frow3)
        pltpu.sync_copy(
            shared_idx.at[lb, pl.ds(0, NPOINT), pl.ds(0, NSAMPLE)], idx_s)

        pending = {}
        for cb in range(CHT // CB):
            c = c0 + cb * CB
            for j in range(CB):
                ch = c + j
                pltpu.sync_copy(
                    features.at[b, ch // 8, ch % 8, pl.ds(0, N // 128),
                                pl.ds(0, 128)],
                    frows[j])

            for rb in range(NPOINT // RB):
                g = cb * (NPOINT // RB) + rb
                cur = g % 2
                if g - 2 in pending:
                    for d in pending.pop(g - 2):
                        d.wait()

                def one_row(r, ri, cur=cur):
                    rv = jnp.full((L,), r, jnp.int32)
                    riv = jnp.full((L,), ri, jnp.int32)
                    for h in range(NSAMPLE // L):
                        cv = h * L + iot
                        iv = plsc.load_gather(idx_s, [rv, cv])
                        ihi = iv >> 7
                        ilo = iv & 127
                        for j in range(CB):
                            vals = plsc.load_gather(frows[j], [ihi, ilo])
                            plsc.store_scatter(
                                obufs2, [jnp.full((L,), cur * CB + j,
                                                  jnp.int32), riv, cv], vals)

                def per_row(ri, carry, rb=rb):
                    r2i = 2 * ri
                    one_row(rb * RB + r2i, r2i)
                    one_row(rb * RB + r2i + 1, r2i + 1)
                    return carry

                lax.fori_loop(0, RB // 2, per_row, 0)
                pending[g] = [
                    pltpu.async_copy(
                        obufs2.at[cur * CB + j, pl.ds(0, RB),
                                  pl.ds(0, NSAMPLE)],
                        out.at[b, 3 + c + j, pl.ds(rb * RB, RB),
                               pl.ds(0, NSAMPLE)],
                        sems.at[cur, j])
                    for j in range(CB)]
        for g in sorted(pending):
            for d in pending[g]:
                d.wait()

    pl.run_scoped(
        phase_b,
        pltpu.VMEM((NPOINT, NSAMPLE), jnp.int32),
        pltpu.VMEM((N // 128, 128), jnp.float32),
        pltpu.VMEM((N // 128, 128), jnp.float32),
        pltpu.VMEM((N // 128, 128), jnp.float32),
        pltpu.VMEM((N // 128, 128), jnp.float32),
        pltpu.VMEM((2 * CB, RB, NSAMPLE), jnp.float32),
        pltpu.SemaphoreType.DMA((2, CB)),
    )


def kernel(xyz, new_xyz, features):
    f5 = features.reshape(B, C // 8, 8, N // 128, 128)
    return _qag_kernel(xyz, new_xyz, f5)


# speedup vs baseline: 2.0025x; 1.1146x over previous
import functools

import jax
import jax.numpy as jnp
from jax import lax
from jax.experimental import pallas as pl
from jax.experimental.pallas import tpu as pltpu
from jax.experimental.pallas import tpu_sc as plsc

B = 4
N = 8192
NPOINT = 1024
NSAMPLE = 32
C = 128
R2 = 0.2 * 0.2

NC = 2
NS = 16
L = 16
NW = NC * NS
CPT = (B * NPOINT) // NW
TPB = NW // B
CHT = C // TPB
NCHUNK = N // L
CB = 4
RB = 128

_mesh = plsc.VectorSubcoreMesh(
    core_axis_name="c", subcore_axis_name="s", num_cores=NC, num_subcores=NS)
_params = pltpu.CompilerParams(use_tc_tiling_on_sc=False,
                               needs_layout_passes=False)


def _bf16_round(x):
    bits = plsc.bitcast(x, jnp.int32)
    lsb = (bits >> 16) & 1
    rb = (bits + 0x7FFF + lsb) & jnp.int32(-65536)
    return plsc.bitcast(rb, jnp.float32)


@functools.partial(
    pl.kernel,
    out_type=jax.ShapeDtypeStruct((B, 3 + C, NPOINT, NSAMPLE), jnp.float32),
    mesh=_mesh,
    compiler_params=_params,
    scratch_types=[
        pltpu.VMEM_SHARED((2, NPOINT, NSAMPLE), jnp.int32),
    ],
)
def _qag_kernel(xyz, new_xyz, features, out, shared_idx):
    ci = lax.axis_index("c")
    si = lax.axis_index("s")
    wid = ci * NS + si
    b = wid // TPB
    lb = b % 2
    m0 = (wid % TPB) * CPT

    iot = lax.iota(jnp.int32, L)
    zz = jnp.zeros((L,), jnp.int32)
    one = zz + 1
    two = zz + 2

    def phase_a(pts, spv, x2b, y2b, z2b, ctrs, selbuf, xyzbuf):
        pltpu.sync_copy(xyz.at[b, pl.ds(0, N), pl.ds(0, 3)], pts)
        pltpu.sync_copy(new_xyz.at[b, pl.ds(m0, CPT), pl.ds(0, 3)], ctrs)

        for w in range(CPT // L):
            plsc.store_scatter(selbuf, [w * L + iot, zz], zz)

        def pre_round(i, carry):
            base = i * L
            bi = base + iot
            xs = plsc.load_gather(pts, [bi, zz])
            ys = plsc.load_gather(pts, [bi, one])
            zs = plsc.load_gather(pts, [bi, two])
            spv[pl.ds(base, L)] = (xs * xs + ys * ys) + zs * zs
            x2b[pl.ds(base, L)] = 2.0 * _bf16_round(xs)
            y2b[pl.ds(base, L)] = 2.0 * _bf16_round(ys)
            z2b[pl.ds(base, L)] = 2.0 * _bf16_round(zs)
            return carry

        lax.fori_loop(0, NCHUNK, pre_round, 0)

        def per_center(mi, carry):
            msp = jnp.full((L,), mi, jnp.int32)
            cx = plsc.load_gather(ctrs, [msp, zz])
            cy = plsc.load_gather(ctrs, [msp, one])
            cz = plsc.load_gather(ctrs, [msp, two])
            sc = (cx * cx + cy * cy) + cz * cz
            cxb = _bf16_round(cx)
            cyb = _bf16_round(cy)
            czb = _bf16_round(cz)

            def cond(st):
                i, cnt = st
                return jnp.logical_and(i < NCHUNK // 4, cnt < NSAMPLE)

            def body(st):
                i, cnt = st
                base0 = i * (4 * L)
                cntv = jnp.full((L,), cnt, jnp.int32)
                run = cntv
                tot = None
                for q in range(4):
                    bq = base0 + q * L
                    d2q = (sc + spv[pl.ds(bq, L)]) - (
                        (cxb * x2b[pl.ds(bq, L)] + cyb * y2b[pl.ds(bq, L)])
                        + czb * z2b[pl.ds(bq, L)])
                    mq = d2q < R2
                    pq = plsc.all_reduce_population_count(mq)
                    incq = plsc.cumsum(mq.astype(jnp.int32))
                    slotq = (run + incq) - 1
                    wmq = jnp.logical_and(mq, slotq < NSAMPLE)
                    plsc.store_scatter(selbuf, [msp, slotq], bq + iot,
                                       mask=wmq)
                    run = run + pq
                    tot = pq if tot is None else tot + pq
                return (i + 1, cnt + tot[0])

            _, cnt = lax.while_loop(cond, body, (jnp.int32(0), jnp.int32(0)))

            firstv = plsc.load_gather(selbuf, [msp, zz])
            cntv = jnp.full((L,), cnt, jnp.int32)
            for h in range(NSAMPLE // L):
                jv = h * L + iot
                cur = plsc.load_gather(selbuf, [msp, jv])
                selv = jnp.where(jv >= cntv, firstv, cur)
                plsc.store_scatter(selbuf, [msp, jv], selv)
                gx = plsc.load_gather(pts, [selv, zz]) - cx
                gy = plsc.load_gather(pts, [selv, one]) - cy
                gz = plsc.load_gather(pts, [selv, two]) - cz
                plsc.store_scatter(xyzbuf, [zz, msp, jv], gx)
                plsc.store_scatter(xyzbuf, [one, msp, jv], gy)
                plsc.store_scatter(xyzbuf, [two, msp, jv], gz)
            return carry

        lax.fori_loop(0, CPT, per_center, 0)

        pltpu.sync_copy(selbuf,
                        shared_idx.at[lb, pl.ds(m0, CPT), pl.ds(0, NSAMPLE)])
        pltpu.sync_copy(
            xyzbuf,
            out.at[b, pl.ds(0, 3), pl.ds(m0, CPT), pl.ds(0, NSAMPLE)])

    pl.run_scoped(
        phase_a,
        pltpu.VMEM((N, 3), jnp.float32),
        pltpu.VMEM((N,), jnp.float32),
        pltpu.VMEM((N,), jnp.float32),
        pltpu.VMEM((N,), jnp.float32),
        pltpu.VMEM((N,), jnp.float32),
        pltpu.VMEM((CPT, 3), jnp.float32),
        pltpu.VMEM((CPT, NSAMPLE), jnp.int32),
        pltpu.VMEM((3, CPT, NSAMPLE), jnp.float32),
    )

    plsc.subcore_barrier()

    c0 = (wid % TPB) * CHT
    jsp = [jnp.full((L,), j, jnp.int32) for j in range(CB)]

    def phase_b(idx_s, frow0, frow1, frow2, frow3, obufs2, sems):
        frows = (frow0, frow1, frow2, frow3)
        pltpu.sync_copy(
            shared_idx.at[lb, pl.ds(0, NPOINT), pl.ds(0, NSAMPLE)], idx_s)

        pending = {}
        for cb in range(CHT // CB):
            c = c0 + cb * CB
            for j in range(CB):
                ch = c + j
                pltpu.sync_copy(
                    features.at[b, ch // 8, ch % 8, pl.ds(0, N // 128),
                                pl.ds(0, 128)],
                    frows[j])

            for rb in range(NPOINT // RB):
                g = cb * (NPOINT // RB) + rb
                cur = g % 2
                if g - 2 in pending:
                    for d in pending.pop(g - 2):
                        d.wait()

                def one_row(r, ri, cur=cur):
                    rv = jnp.full((L,), r, jnp.int32)
                    riv = jnp.full((L,), ri, jnp.int32)
                    for h in range(NSAMPLE // L):
                        cv = h * L + iot
                        iv = plsc.load_gather(idx_s, [rv, cv])
                        ihi = iv >> 7
                        ilo = iv & 127
                        for j in range(CB):
                            vals = plsc.load_gather(frows[j], [ihi, ilo])
                            plsc.store_scatter(
                                obufs2, [jnp.full((L,), cur * CB + j,
                                                  jnp.int32), riv, cv], vals)

                def per_row(ri, carry, rb=rb):
                    r2i = 2 * ri
                    one_row(rb * RB + r2i, r2i)
                    one_row(rb * RB + r2i + 1, r2i + 1)
                    return carry

                lax.fori_loop(0, RB // 2, per_row, 0)
                pending[g] = [
                    pltpu.async_copy(
                        obufs2.at[cur * CB + j, pl.ds(0, RB),
                                  pl.ds(0, NSAMPLE)],
                        out.at[b, 3 + c + j, pl.ds(rb * RB, RB),
                               pl.ds(0, NSAMPLE)],
                        sems.at[cur, j])
                    for j in range(CB)]
        for g in sorted(pending):
            for d in pending[g]:
                d.wait()

    pl.run_scoped(
        phase_b,
        pltpu.VMEM((NPOINT, NSAMPLE), jnp.int32),
        pltpu.VMEM((N // 128, 128), jnp.float32),
        pltpu.VMEM((N // 128, 128), jnp.float32),
        pltpu.VMEM((N // 128, 128), jnp.float32),
        pltpu.VMEM((N // 128, 128), jnp.float32),
        pltpu.VMEM((2 * CB, RB, NSAMPLE), jnp.float32),
        pltpu.SemaphoreType.DMA((2, CB)),
    )


def kernel(xyz, new_xyz, features):
    f5 = features.reshape(B, C // 8, 8, N // 128, 128)
    return _qag_kernel(xyz, new_xyz, f5)
